# baseline (speedup 1.0000x reference)
import jax
import jax.numpy as jnp
from jax import lax
from jax.experimental import pallas as pl
from jax.experimental.pallas import tpu as pltpu

_B, _C, _H, _W = 16, 256, 32, 32
_HW = _H * _W
_NCHUNK = 8
_RPC = _HW // _NCHUNK
_HPC = _H // _NCHUNK


def _pos_kernel(row_ref, col_ref, out_ref, plane, sem):
    col = col_ref[:_W, :]
    row = row_ref[:_H, :]
    copies = []
    for q in range(_NCHUNK):
        r0 = q * _RPC
        plane[pl.ds(r0, _RPC), :_C] = jnp.broadcast_to(
            col[None, :, :], (_HPC, _W, _C)).reshape(_RPC, _C)
        plane[pl.ds(r0, _RPC), _C:] = jnp.broadcast_to(
            row[q * _HPC:(q + 1) * _HPC, None, :], (_HPC, _W, _C)
        ).reshape(_RPC, _C)
        chunk = plane.at[pl.ds(r0, _RPC)]
        for b in range(_B):
            cp = pltpu.make_async_copy(
                chunk, out_ref.at[b, pl.ds(r0, _RPC)], sem)
            cp.start()
            copies.append(cp)
    for cp in copies:
        cp.wait()


def kernel(mask, row_embed, col_embed):
    B, H, W = mask.shape
    C = row_embed.shape[1]
    out = pl.pallas_call(
        _pos_kernel,
        in_specs=[
            pl.BlockSpec(memory_space=pltpu.VMEM),
            pl.BlockSpec(memory_space=pltpu.VMEM),
        ],
        out_specs=pl.BlockSpec(memory_space=pl.ANY),
        out_shape=jax.ShapeDtypeStruct((B, H * W, 2 * C), jnp.float32),
        scratch_shapes=[
            pltpu.VMEM((H * W, 2 * C), jnp.float32),
            pltpu.SemaphoreType.DMA,
        ],
    )(row_embed, col_embed)
    return out.reshape(B, H, W, 2 * C).transpose(0, 3, 1, 2)


# speedup vs baseline: 1.0094x; 1.0094x over previous
import jax
import jax.numpy as jnp
from jax.experimental import pallas as pl
from jax.experimental.pallas import tpu as pltpu

_B, _C, _H, _W = 16, 256, 32, 32
_HW = _H * _W
_NCHUNK = 4
_RPC = _HW // _NCHUNK
_HPC = _H // _NCHUNK


def _pos_kernel(row_ref, col_ref, out_ref, plane, sem):
    col = col_ref[:_W, :]
    row = row_ref[:_H, :]
    copies = []
    for q in range(_NCHUNK):
        r0 = q * _RPC
        plane[pl.ds(r0, _RPC), :_C] = jnp.broadcast_to(
            col[None, :, :], (_HPC, _W, _C)).reshape(_RPC, _C)
        plane[pl.ds(r0, _RPC), _C:] = jnp.broadcast_to(
            row[q * _HPC:(q + 1) * _HPC, None, :], (_HPC, _W, _C)
        ).reshape(_RPC, _C)
        chunk = plane.at[pl.ds(r0, _RPC)]
        for b in range(_B):
            cp = pltpu.make_async_copy(
                chunk, out_ref.at[b, pl.ds(r0, _RPC)], sem)
            cp.start()
            copies.append(cp)
    for cp in copies:
        cp.wait()


def kernel(mask, row_embed, col_embed):
    B, H, W = mask.shape
    C = row_embed.shape[1]
    out = pl.pallas_call(
        _pos_kernel,
        in_specs=[
            pl.BlockSpec(memory_space=pltpu.VMEM),
            pl.BlockSpec(memory_space=pltpu.VMEM),
        ],
        out_specs=pl.BlockSpec(memory_space=pl.ANY),
        out_shape=jax.ShapeDtypeStruct((B, H * W, 2 * C), jnp.float32),
        scratch_shapes=[
            pltpu.VMEM((H * W, 2 * C), jnp.float32),
            pltpu.SemaphoreType.DMA,
        ],
    )(row_embed, col_embed)
    return out.reshape(B, H, W, 2 * C).transpose(0, 3, 1, 2)
